# baseline (speedup 1.0000x reference)
import functools

import jax
import jax.numpy as jnp
from jax.experimental import pallas as pl
from jax.experimental.pallas import tpu as pltpu

_LAYERS = [(32, 3, True), (64, 3, True), (128, 3, False), (64, 1, False),
           (128, 3, True), (256, 3, False), (128, 1, False), (256, 3, True),
           (512, 3, False), (256, 1, False), (512, 3, False), (256, 1, False),
           (512, 3, True), (1024, 3, False), (512, 1, False), (1024, 3, False),
           (512, 1, False), (1024, 3, False)]
_CLS = 20
_BB = 2
_S = 7


def _leaky(y):
    return jnp.where(y >= 0, y, 0.1 * y)


def _shiftmax_lane(m):
    return jnp.maximum(m, jnp.concatenate([m[:, 1:], m[:, :1]], axis=1))


def _hcw_body(x_ref, w_ref, s_ref, b_ref, o_ref, *, H, W, C, O, K, pool):
    o_ref[...] = jnp.zeros_like(o_ref)
    w = w_ref[...]
    scale = s_ref[...]
    bias = b_ref[...]

    def row(h):
        if K == 3:
            rows = [x_ref[0, h + a] for a in (0, 1, 2)]
            patch = jnp.concatenate(
                [r[:, b:b + W] for r in rows for b in (0, 1, 2)], axis=0)
        else:
            patch = x_ref[0, h + 1][:, 1:1 + W]
        y = jax.lax.dot_general(w, patch, (((1,), (0,)), ((), ())),
                                preferred_element_type=jnp.float32,
                                precision=jax.lax.Precision.HIGHEST)
        return _leaky(y * scale + bias)

    if pool:
        def body(hp, c):
            m = jnp.maximum(row(2 * hp), row(2 * hp + 1))
            o_ref[0, hp + 1, :, 1:1 + W] = _shiftmax_lane(m)
            return c
        jax.lax.fori_loop(0, H // 2, body, 0)
    else:
        def body(h, c):
            o_ref[0, h + 1, :, 1:1 + W] = row(h)
            return c
        jax.lax.fori_loop(0, H, body, 0)


def _conv_hcw(x, w_mat, s, b, *, H, W, C, O, K, pool):
    N = x.shape[0]
    Ho = (H // 2) if pool else H
    out_shape = (N, Ho + 2, O, W + 2)
    body = functools.partial(_hcw_body, H=H, W=W, C=C, O=O, K=K, pool=pool)
    return pl.pallas_call(
        body,
        grid=(N,),
        in_specs=[
            pl.BlockSpec((1, H + 2, C, W + 2), lambda n: (n, 0, 0, 0)),
            pl.BlockSpec(w_mat.shape, lambda n: (0, 0)),
            pl.BlockSpec((O, 1), lambda n: (0, 0)),
            pl.BlockSpec((O, 1), lambda n: (0, 0)),
        ],
        out_specs=pl.BlockSpec((1,) + out_shape[1:], lambda n: (n, 0, 0, 0)),
        out_shape=jax.ShapeDtypeStruct(out_shape, jnp.float32),
    )(x, w_mat, s[:, None], b[:, None])


def _nhwc_body(x_ref, w_ref, s_ref, b_ref, o_ref, *, N, H, W, C, O, K,
               pool, G):
    o_ref[...] = jnp.zeros_like(o_ref)
    w = w_ref[...]
    scale = s_ref[...]
    bias = b_ref[...]
    ng = H // G

    def body(i, c):
        n = i // ng
        g = i % ng
        h0 = g * G
        if K == 3:
            parts = []
            for a in (0, 1, 2):
                xs = x_ref[n, pl.ds(h0 + a, G), :, :]
                for b2 in (0, 1, 2):
                    parts.append(xs[:, b2:b2 + W, :])
            patch = jnp.concatenate(parts, axis=-1)
        else:
            patch = x_ref[n, pl.ds(h0 + 1, G), 1:1 + W, :]
        patch = patch.reshape(G * W, patch.shape[-1])
        y = jax.lax.dot_general(patch, w, (((1,), (0,)), ((), ())),
                                preferred_element_type=jnp.float32,
                                precision=jax.lax.Precision.HIGHEST)
        y = _leaky(y * scale + bias).reshape(G, W, O)
        if pool:
            y2 = y.reshape(G // 2, 2, W, O)
            m = jnp.maximum(y2[:, 0], y2[:, 1])
            ms = jnp.maximum(
                m, jnp.concatenate([m[:, 1:, :], m[:, :1, :]], axis=1))
            o_ref[n, pl.ds(g * (G // 2) + 1, G // 2), 1:1 + W, :] = ms
        else:
            o_ref[n, pl.ds(g * G + 1, G), 1:1 + W, :] = y
        return c

    jax.lax.fori_loop(0, N * ng, body, 0)


def _conv_nhwc(x, w_mat, s, b, *, H, W, C, O, K, pool, G):
    N = x.shape[0]
    Ho = (H // 2) if pool else H
    out_shape = (N, Ho + 2, W + 2, O)
    body = functools.partial(_nhwc_body, N=N, H=H, W=W, C=C, O=O, K=K,
                             pool=pool, G=G)
    return pl.pallas_call(
        body,
        in_specs=[
            pl.BlockSpec(x.shape, lambda: (0,) * 4),
            pl.BlockSpec(w_mat.shape, lambda: (0, 0)),
            pl.BlockSpec((1, O), lambda: (0, 0)),
            pl.BlockSpec((1, O), lambda: (0, 0)),
        ],
        out_specs=pl.BlockSpec(out_shape, lambda: (0,) * 4),
        out_shape=jax.ShapeDtypeStruct(out_shape, jnp.float32),
    )(x, w_mat, s, b)


def _local_body(*refs, N, stride):
    if stride == 2:
        x00, x01, x10, x11, w_ref, s_ref, b_ref, o_ref = refs
        xp = ((x00, x01), (x10, x11))
    else:
        x_ref, w_ref, s_ref, b_ref, o_ref = refs
    o_ref[...] = jnp.zeros_like(o_ref)
    w = w_ref[...]
    scale = s_ref[...]
    bias = b_ref[...]
    parts = []
    for a in (0, 1, 2):
        for b2 in (0, 1, 2):
            if stride == 2:
                xs = xp[a % 2][b2 % 2][:, a // 2:a // 2 + 7,
                                       b2 // 2:b2 // 2 + 7, :]
            else:
                xs = x_ref[:, a:a + 7, b2:b2 + 7, :]
            parts.append(xs.reshape(N * 49, 1024))
    patch = jnp.concatenate(parts, axis=-1)
    y = jax.lax.dot_general(patch, w, (((1,), (0,)), ((), ())),
                            preferred_element_type=jnp.float32,
                                precision=jax.lax.Precision.HIGHEST)
    y = _leaky(y * scale + bias).reshape(N, 7, 7, 1024)
    o_ref[:, 1:8, 1:8, :] = y


def _conv_local(xs, w_mat, s, b, *, stride):
    N = xs[0].shape[0]
    out_shape = (N, 9, 9, 1024)
    body = functools.partial(_local_body, N=N, stride=stride)
    in_arrays = list(xs) + [w_mat, s, b]
    return pl.pallas_call(
        body,
        in_specs=[pl.BlockSpec(a.shape, (lambda nd=a.ndim: (0,) * nd))
                  for a in in_arrays],
        out_specs=pl.BlockSpec(out_shape, lambda: (0,) * 4),
        out_shape=jax.ShapeDtypeStruct(out_shape, jnp.float32),
    )(*in_arrays)


def _fc_body(w_ref, x_ref, b_ref, o_ref, *, nk):
    k = pl.program_id(0)

    @pl.when(k == 0)
    def _():
        o_ref[...] = jnp.zeros_like(o_ref)

    o_ref[...] += jnp.dot(w_ref[...], x_ref[...],
                          preferred_element_type=jnp.float32,
                                precision=jax.lax.Precision.HIGHEST)

    @pl.when(k == nk - 1)
    def _():
        o_ref[...] = _leaky(o_ref[...] + b_ref[...])


def _fc_reg(w, xT, bias):
    KB = 1024
    nk = w.shape[1] // KB
    return pl.pallas_call(
        functools.partial(_fc_body, nk=nk),
        grid=(nk,),
        in_specs=[
            pl.BlockSpec((4096, KB), lambda k: (0, k)),
            pl.BlockSpec((KB, 8), lambda k: (k, 0)),
            pl.BlockSpec((4096, 1), lambda k: (0, 0)),
        ],
        out_specs=pl.BlockSpec((4096, 8), lambda k: (0, 0)),
        out_shape=jax.ShapeDtypeStruct((4096, 8), jnp.float32),
    )(w, xT, bias)


def _heads_body(h_ref, cw_ref, cb_ref, rw_ref, rb_ref, ow_ref, ob_ref,
                oc_ref, orr_ref, oo_ref):
    h = h_ref[...]
    oc_ref[...] = jnp.dot(cw_ref[...], h,
                          preferred_element_type=jnp.float32,
                                precision=jax.lax.Precision.HIGHEST) + cb_ref[...]
    orr_ref[...] = jnp.dot(rw_ref[...], h,
                           preferred_element_type=jnp.float32,
                                precision=jax.lax.Precision.HIGHEST) + rb_ref[...]
    oo_ref[...] = jnp.dot(ow_ref[...], h,
                          preferred_element_type=jnp.float32,
                                precision=jax.lax.Precision.HIGHEST) + ob_ref[...]


def _heads(h8, cw, cb, rw, rb, ow, ob):
    args = (h8, cw, cb, rw, rb, ow, ob)
    specs = [pl.BlockSpec(a.shape, lambda: (0, 0)) for a in args]
    return pl.pallas_call(
        _heads_body,
        in_specs=specs,
        out_specs=[pl.BlockSpec((cw.shape[0], 8), lambda: (0, 0)),
                   pl.BlockSpec((rw.shape[0], 8), lambda: (0, 0)),
                   pl.BlockSpec((ow.shape[0], 8), lambda: (0, 0))],
        out_shape=[jax.ShapeDtypeStruct((cw.shape[0], 8), jnp.float32),
                   jax.ShapeDtypeStruct((rw.shape[0], 8), jnp.float32),
                   jax.ShapeDtypeStruct((ow.shape[0], 8), jnp.float32)],
    )(*args)


def _affine(p):
    s = p['gamma'] * jax.lax.rsqrt(p['var'] + 1e-5)
    b = p['beta'] - p['mean'] * s
    return s, b


def kernel(x, target, params):
    del target
    N = x.shape[0]
    dk = params['darknet']

    out = jnp.pad(x.transpose(0, 2, 1, 3), ((0, 0), (1, 1), (0, 0), (1, 1)))

    sizes = [448, 224, 112, 112, 112]
    cins = [3, 32, 64, 128, 64]
    for i in range(5):
        O, K, pool = _LAYERS[i]
        C, H = cins[i], sizes[i]
        p = dk[i]
        s, b = _affine(p)
        if K == 3:
            w_mat = p['w'].transpose(0, 2, 3, 1).reshape(O, 9 * C)
        else:
            w_mat = p['w'].reshape(O, C)
        out = _conv_hcw(out, w_mat, s, b, H=H, W=H, C=C, O=O, K=K, pool=pool)
        if pool and i < 4:
            out = jnp.pad(out[:, :, :, 1:1 + H:2],
                          ((0, 0), (0, 0), (0, 0), (1, 1)))
    out = jnp.pad(out[:, :, :, 1:113:2].transpose(0, 1, 3, 2),
                  ((0, 0), (0, 0), (1, 1), (0, 0)))

    sizes = [56, 56, 56, 28, 28, 28, 28, 28, 14, 14, 14, 14, 14]
    cins = [128, 256, 128, 256, 512, 256, 512, 256, 512, 1024, 512, 1024, 512]
    groups = {56: 8, 28: 14, 14: 14}
    for i in range(5, 18):
        O, K, pool = _LAYERS[i]
        C, H = cins[i - 5], sizes[i - 5]
        p = dk[i]
        s, b = _affine(p)
        if K == 3:
            w_mat = p['w'].transpose(2, 3, 1, 0).reshape(9 * C, O)
        else:
            w_mat = p['w'].reshape(O, C).T
        out = _conv_nhwc(out, w_mat, s[None, :], b[None, :], H=H, W=H, C=C,
                         O=O, K=K, pool=pool, G=groups[H])
        if pool:
            out = jnp.pad(out[:, :, 1:1 + H:2, :],
                          ((0, 0), (0, 0), (1, 1), (0, 0)))

    strides = [2, 1, 1, 1]
    for i in range(4):
        p = params['local'][i]
        s, b = _affine(p)
        w_mat = p['w'].transpose(2, 3, 1, 0).reshape(9 * 1024, 1024)
        if strides[i] == 2:
            xs = [out[:, pa::2, pb::2, :] for pa in (0, 1) for pb in (0, 1)]
        else:
            xs = [out]
        out = _conv_local(xs, w_mat, s[None, :], b[None, :], stride=strides[i])

    act = out[:, 1:8, 1:8, :]
    flatT = act.transpose(3, 1, 2, 0).reshape(1024 * 49, N)
    flatT8 = jnp.pad(flatT, ((0, 0), (0, 8 - N)))
    h8 = _fc_reg(params['reg_W'], flatT8, params['reg_b'][:, None])
    clsT, respT, offT = _heads(h8,
                               params['cls_W'], params['cls_b'][:, None],
                               params['resp_W'], params['resp_b'][:, None],
                               params['off_W'], params['off_b'][:, None])
    pred_cls = clsT[:, :N].T.reshape(N, _CLS, _S, _S)
    pred_resp = respT[:, :N].T.reshape(N, _BB, _S, _S)
    pred_bbox = offT[:, :N].T.reshape(N, _BB * 4, _S, _S)
    return (pred_cls, pred_resp, pred_bbox)


# speedup vs baseline: 1.4547x; 1.4547x over previous
import functools

import jax
import jax.numpy as jnp
from jax.experimental import pallas as pl
from jax.experimental.pallas import tpu as pltpu

_LAYERS = [(32, 3, True), (64, 3, True), (128, 3, False), (64, 1, False),
           (128, 3, True), (256, 3, False), (128, 1, False), (256, 3, True),
           (512, 3, False), (256, 1, False), (512, 3, False), (256, 1, False),
           (512, 3, True), (1024, 3, False), (512, 1, False), (1024, 3, False),
           (512, 1, False), (1024, 3, False)]
_CLS = 20
_BB = 2
_S = 7


def _leaky(y):
    return jnp.where(y >= 0, y, 0.1 * y)


def _shiftmax_lane(m):
    return jnp.maximum(m, jnp.concatenate([m[:, 1:], m[:, :1]], axis=1))


def _hcw_body(x_ref, w_ref, s_ref, b_ref, o_ref, *, H, W, C, O, K, pool):
    o_ref[...] = jnp.zeros_like(o_ref)
    w = w_ref[...]
    scale = s_ref[...]
    bias = b_ref[...]

    def row(h):
        if K == 3:
            rows = [x_ref[0, h + a] for a in (0, 1, 2)]
            patch = jnp.concatenate(
                [r[:, b:b + W] for r in rows for b in (0, 1, 2)], axis=0)
        else:
            patch = x_ref[0, h + 1][:, 1:1 + W]
        y = jax.lax.dot_general(w, patch, (((1,), (0,)), ((), ())),
                                preferred_element_type=jnp.float32)
        return _leaky(y * scale + bias)

    if pool:
        def body(hp, c):
            m = jnp.maximum(row(2 * hp), row(2 * hp + 1))
            o_ref[0, hp + 1, :, 1:1 + W] = _shiftmax_lane(m)
            return c
        jax.lax.fori_loop(0, H // 2, body, 0)
    else:
        def body(h, c):
            o_ref[0, h + 1, :, 1:1 + W] = row(h)
            return c
        jax.lax.fori_loop(0, H, body, 0)


def _conv_hcw(x, w_mat, s, b, *, H, W, C, O, K, pool):
    N = x.shape[0]
    Ho = (H // 2) if pool else H
    out_shape = (N, Ho + 2, O, W + 2)
    body = functools.partial(_hcw_body, H=H, W=W, C=C, O=O, K=K, pool=pool)
    return pl.pallas_call(
        body,
        grid=(N,),
        in_specs=[
            pl.BlockSpec((1, H + 2, C, W + 2), lambda n: (n, 0, 0, 0)),
            pl.BlockSpec(w_mat.shape, lambda n: (0, 0)),
            pl.BlockSpec((O, 1), lambda n: (0, 0)),
            pl.BlockSpec((O, 1), lambda n: (0, 0)),
        ],
        out_specs=pl.BlockSpec((1,) + out_shape[1:], lambda n: (n, 0, 0, 0)),
        out_shape=jax.ShapeDtypeStruct(out_shape, jnp.float32),
    )(x, w_mat, s[:, None], b[:, None])


def _nhwc_body(x_ref, w_ref, s_ref, b_ref, o_ref, *, N, H, W, C, O, K,
               pool, G):
    o_ref[...] = jnp.zeros_like(o_ref)
    w = w_ref[...]
    scale = s_ref[...]
    bias = b_ref[...]
    ng = H // G

    def body(i, c):
        n = i // ng
        g = i % ng
        h0 = g * G
        if K == 3:
            parts = []
            for a in (0, 1, 2):
                xs = x_ref[n, pl.ds(h0 + a, G), :, :]
                for b2 in (0, 1, 2):
                    parts.append(xs[:, b2:b2 + W, :])
            patch = jnp.concatenate(parts, axis=-1)
        else:
            patch = x_ref[n, pl.ds(h0 + 1, G), 1:1 + W, :]
        patch = patch.reshape(G * W, patch.shape[-1])
        y = jax.lax.dot_general(patch, w, (((1,), (0,)), ((), ())),
                                preferred_element_type=jnp.float32)
        y = _leaky(y * scale + bias).reshape(G, W, O)
        if pool:
            y2 = y.reshape(G // 2, 2, W, O)
            m = jnp.maximum(y2[:, 0], y2[:, 1])
            ms = jnp.maximum(
                m, jnp.concatenate([m[:, 1:, :], m[:, :1, :]], axis=1))
            o_ref[n, pl.ds(g * (G // 2) + 1, G // 2), 1:1 + W, :] = ms
        else:
            o_ref[n, pl.ds(g * G + 1, G), 1:1 + W, :] = y
        return c

    jax.lax.fori_loop(0, N * ng, body, 0)


def _conv_nhwc(x, w_mat, s, b, *, H, W, C, O, K, pool, G):
    N = x.shape[0]
    Ho = (H // 2) if pool else H
    out_shape = (N, Ho + 2, W + 2, O)
    body = functools.partial(_nhwc_body, N=N, H=H, W=W, C=C, O=O, K=K,
                             pool=pool, G=G)
    return pl.pallas_call(
        body,
        in_specs=[
            pl.BlockSpec(x.shape, lambda: (0,) * 4),
            pl.BlockSpec(w_mat.shape, lambda: (0, 0)),
            pl.BlockSpec((1, O), lambda: (0, 0)),
            pl.BlockSpec((1, O), lambda: (0, 0)),
        ],
        out_specs=pl.BlockSpec(out_shape, lambda: (0,) * 4),
        out_shape=jax.ShapeDtypeStruct(out_shape, jnp.float32),
    )(x, w_mat, s, b)


def _local_body(*refs, N, stride):
    if stride == 2:
        x00, x01, x10, x11, w_ref, s_ref, b_ref, o_ref = refs
        xp = ((x00, x01), (x10, x11))
    else:
        x_ref, w_ref, s_ref, b_ref, o_ref = refs
    o_ref[...] = jnp.zeros_like(o_ref)
    w = w_ref[...]
    scale = s_ref[...]
    bias = b_ref[...]
    parts = []
    for a in (0, 1, 2):
        for b2 in (0, 1, 2):
            if stride == 2:
                xs = xp[a % 2][b2 % 2][:, a // 2:a // 2 + 7,
                                       b2 // 2:b2 // 2 + 7, :]
            else:
                xs = x_ref[:, a:a + 7, b2:b2 + 7, :]
            parts.append(xs.reshape(N * 49, 1024))
    patch = jnp.concatenate(parts, axis=-1)
    y = jax.lax.dot_general(patch, w, (((1,), (0,)), ((), ())),
                            preferred_element_type=jnp.float32)
    y = _leaky(y * scale + bias).reshape(N, 7, 7, 1024)
    o_ref[:, 1:8, 1:8, :] = y


def _conv_local(xs, w_mat, s, b, *, stride):
    N = xs[0].shape[0]
    out_shape = (N, 9, 9, 1024)
    body = functools.partial(_local_body, N=N, stride=stride)
    in_arrays = list(xs) + [w_mat, s, b]
    return pl.pallas_call(
        body,
        in_specs=[pl.BlockSpec(a.shape, (lambda nd=a.ndim: (0,) * nd))
                  for a in in_arrays],
        out_specs=pl.BlockSpec(out_shape, lambda: (0,) * 4),
        out_shape=jax.ShapeDtypeStruct(out_shape, jnp.float32),
    )(*in_arrays)


def _fc_body(w_ref, x_ref, b_ref, o_ref, *, nk):
    k = pl.program_id(0)

    @pl.when(k == 0)
    def _():
        o_ref[...] = jnp.zeros_like(o_ref)

    o_ref[...] += jnp.dot(w_ref[...], x_ref[...],
                          preferred_element_type=jnp.float32)

    @pl.when(k == nk - 1)
    def _():
        o_ref[...] = _leaky(o_ref[...] + b_ref[...])


def _fc_reg(w, xT, bias):
    KB = 1024
    nk = w.shape[1] // KB
    return pl.pallas_call(
        functools.partial(_fc_body, nk=nk),
        grid=(nk,),
        in_specs=[
            pl.BlockSpec((4096, KB), lambda k: (0, k)),
            pl.BlockSpec((KB, 8), lambda k: (k, 0)),
            pl.BlockSpec((4096, 1), lambda k: (0, 0)),
        ],
        out_specs=pl.BlockSpec((4096, 8), lambda k: (0, 0)),
        out_shape=jax.ShapeDtypeStruct((4096, 8), jnp.float32),
    )(w, xT, bias)


def _heads_body(h_ref, cw_ref, cb_ref, rw_ref, rb_ref, ow_ref, ob_ref,
                oc_ref, orr_ref, oo_ref):
    h = h_ref[...]
    oc_ref[...] = jnp.dot(cw_ref[...], h,
                          preferred_element_type=jnp.float32) + cb_ref[...]
    orr_ref[...] = jnp.dot(rw_ref[...], h,
                           preferred_element_type=jnp.float32) + rb_ref[...]
    oo_ref[...] = jnp.dot(ow_ref[...], h,
                          preferred_element_type=jnp.float32) + ob_ref[...]


def _heads(h8, cw, cb, rw, rb, ow, ob):
    args = (h8, cw, cb, rw, rb, ow, ob)
    specs = [pl.BlockSpec(a.shape, lambda: (0, 0)) for a in args]
    return pl.pallas_call(
        _heads_body,
        in_specs=specs,
        out_specs=[pl.BlockSpec((cw.shape[0], 8), lambda: (0, 0)),
                   pl.BlockSpec((rw.shape[0], 8), lambda: (0, 0)),
                   pl.BlockSpec((ow.shape[0], 8), lambda: (0, 0))],
        out_shape=[jax.ShapeDtypeStruct((cw.shape[0], 8), jnp.float32),
                   jax.ShapeDtypeStruct((rw.shape[0], 8), jnp.float32),
                   jax.ShapeDtypeStruct((ow.shape[0], 8), jnp.float32)],
    )(*args)


def _affine(p):
    s = p['gamma'] * jax.lax.rsqrt(p['var'] + 1e-5)
    b = p['beta'] - p['mean'] * s
    return s, b


def kernel(x, target, params):
    del target
    N = x.shape[0]
    dk = params['darknet']

    out = jnp.pad(x.transpose(0, 2, 1, 3), ((0, 0), (1, 1), (0, 0), (1, 1)))

    sizes = [448, 224, 112, 112, 112]
    cins = [3, 32, 64, 128, 64]
    for i in range(5):
        O, K, pool = _LAYERS[i]
        C, H = cins[i], sizes[i]
        p = dk[i]
        s, b = _affine(p)
        if K == 3:
            w_mat = p['w'].transpose(0, 2, 3, 1).reshape(O, 9 * C)
        else:
            w_mat = p['w'].reshape(O, C)
        out = _conv_hcw(out, w_mat, s, b, H=H, W=H, C=C, O=O, K=K, pool=pool)
        if pool and i < 4:
            out = jnp.pad(out[:, :, :, 1:1 + H:2],
                          ((0, 0), (0, 0), (0, 0), (1, 1)))
    out = jnp.pad(out[:, :, :, 1:113:2].transpose(0, 1, 3, 2),
                  ((0, 0), (0, 0), (1, 1), (0, 0)))

    sizes = [56, 56, 56, 28, 28, 28, 28, 28, 14, 14, 14, 14, 14]
    cins = [128, 256, 128, 256, 512, 256, 512, 256, 512, 1024, 512, 1024, 512]
    groups = {56: 8, 28: 14, 14: 14}
    for i in range(5, 18):
        O, K, pool = _LAYERS[i]
        C, H = cins[i - 5], sizes[i - 5]
        p = dk[i]
        s, b = _affine(p)
        if K == 3:
            w_mat = p['w'].transpose(2, 3, 1, 0).reshape(9 * C, O)
        else:
            w_mat = p['w'].reshape(O, C).T
        out = _conv_nhwc(out, w_mat, s[None, :], b[None, :], H=H, W=H, C=C,
                         O=O, K=K, pool=pool, G=groups[H])
        if pool:
            out = jnp.pad(out[:, :, 1:1 + H:2, :],
                          ((0, 0), (0, 0), (1, 1), (0, 0)))

    strides = [2, 1, 1, 1]
    for i in range(4):
        p = params['local'][i]
        s, b = _affine(p)
        w_mat = p['w'].transpose(2, 3, 1, 0).reshape(9 * 1024, 1024)
        if strides[i] == 2:
            xs = [out[:, pa::2, pb::2, :] for pa in (0, 1) for pb in (0, 1)]
        else:
            xs = [out]
        out = _conv_local(xs, w_mat, s[None, :], b[None, :], stride=strides[i])

    act = out[:, 1:8, 1:8, :]
    flatT = act.transpose(3, 1, 2, 0).reshape(1024 * 49, N)
    flatT8 = jnp.pad(flatT, ((0, 0), (0, 8 - N)))
    h8 = _fc_reg(params['reg_W'], flatT8, params['reg_b'][:, None])
    clsT, respT, offT = _heads(h8,
                               params['cls_W'], params['cls_b'][:, None],
                               params['resp_W'], params['resp_b'][:, None],
                               params['off_W'], params['off_b'][:, None])
    pred_cls = clsT[:, :N].T.reshape(N, _CLS, _S, _S)
    pred_resp = respT[:, :N].T.reshape(N, _BB, _S, _S)
    pred_bbox = offT[:, :N].T.reshape(N, _BB * 4, _S, _S)
    return (pred_cls, pred_resp, pred_bbox)


# speedup vs baseline: 1.6569x; 1.1390x over previous
import functools

import jax
import jax.numpy as jnp
from jax.experimental import pallas as pl
from jax.experimental.pallas import tpu as pltpu

_LAYERS = [(32, 3, True), (64, 3, True), (128, 3, False), (64, 1, False),
           (128, 3, True), (256, 3, False), (128, 1, False), (256, 3, True),
           (512, 3, False), (256, 1, False), (512, 3, False), (256, 1, False),
           (512, 3, True), (1024, 3, False), (512, 1, False), (1024, 3, False),
           (512, 1, False), (1024, 3, False)]
_CLS = 20
_BB = 2
_S = 7


def _leaky(y):
    return jnp.where(y >= 0, y, 0.1 * y)


def _bn_leaky(y, m, p, g, b):
    return _leaky((y - m) / p * g + b)


def _shiftmax_lane(m):
    return jnp.maximum(m, jnp.concatenate([m[:, 1:], m[:, :1]], axis=1))


def _hcw_body(x_ref, w_ref, m_ref, p_ref, g_ref, bb_ref, o_ref, *, H, W, C, O, K, pool, R):
    o_ref[...] = jnp.zeros_like(o_ref)
    w = w_ref[...]
    bn = (m_ref[...], p_ref[...], g_ref[...], bb_ref[...])

    def group(h0):
        if K == 3:
            xr = [x_ref[0, h0 + j] for j in range(R + 2)]
            patch = jnp.concatenate(
                [jnp.concatenate([xr[r + a][:, b:b + W] for r in range(R)],
                                 axis=1)
                 for a in (0, 1, 2) for b in (0, 1, 2)], axis=0)
        else:
            patch = jnp.concatenate(
                [x_ref[0, h0 + 1 + r][:, 1:1 + W] for r in range(R)], axis=1)
        y = jax.lax.dot_general(w, patch, (((1,), (0,)), ((), ())),
                                preferred_element_type=jnp.float32)
        return _bn_leaky(y, *bn)

    if pool:
        def body(i, c):
            y = group(i * R)
            for k in range(R // 2):
                m = jnp.maximum(y[:, 2 * k * W:(2 * k + 1) * W],
                                y[:, (2 * k + 1) * W:(2 * k + 2) * W])
                o_ref[0, i * (R // 2) + k + 1, :, 1:1 + W] = _shiftmax_lane(m)
            return c
        jax.lax.fori_loop(0, H // R, body, 0)
    else:
        def body(i, c):
            y = group(i * R)
            for r in range(R):
                o_ref[0, i * R + r + 1, :, 1:1 + W] = y[:, r * W:(r + 1) * W]
            return c
        jax.lax.fori_loop(0, H // R, body, 0)


def _conv_hcw(x, w_mat, bn, *, H, W, C, O, K, pool):
    N = x.shape[0]
    Ho = (H // 2) if pool else H
    out_shape = (N, Ho + 2, O, W + 2)
    body = functools.partial(_hcw_body, H=H, W=W, C=C, O=O, K=K, pool=pool,
                             R=8)
    return pl.pallas_call(
        body,
        grid=(N,),
        in_specs=[
            pl.BlockSpec((1, H + 2, C, W + 2), lambda n: (n, 0, 0, 0)),
            pl.BlockSpec(w_mat.shape, lambda n: (0, 0)),
        ] + [pl.BlockSpec((O, 1), lambda n: (0, 0))] * 4,
        out_specs=pl.BlockSpec((1,) + out_shape[1:], lambda n: (n, 0, 0, 0)),
        out_shape=jax.ShapeDtypeStruct(out_shape, jnp.float32),
    )(x, w_mat, *[v[:, None] for v in bn])


def _nhwc_body(x_ref, w_ref, m_ref, p_ref, g_ref, bb_ref, o_ref, *, N, H, W, C, O, K,
               pool, G):
    o_ref[...] = jnp.zeros_like(o_ref)
    w = w_ref[...]
    bn = (m_ref[...], p_ref[...], g_ref[...], bb_ref[...])
    ng = H // G

    def body(i, c):
        n = i // ng
        g = i % ng
        h0 = g * G
        if K == 3:
            parts = []
            for a in (0, 1, 2):
                xs = x_ref[n, pl.ds(h0 + a, G), :, :]
                for b2 in (0, 1, 2):
                    parts.append(xs[:, b2:b2 + W, :])
            patch = jnp.concatenate(parts, axis=-1)
        else:
            patch = x_ref[n, pl.ds(h0 + 1, G), 1:1 + W, :]
        patch = patch.reshape(G * W, patch.shape[-1])
        y = jax.lax.dot_general(patch, w, (((1,), (0,)), ((), ())),
                                preferred_element_type=jnp.float32)
        y = _bn_leaky(y, *bn).reshape(G, W, O)
        if pool:
            y2 = y.reshape(G // 2, 2, W, O)
            m = jnp.maximum(y2[:, 0], y2[:, 1])
            ms = jnp.maximum(
                m, jnp.concatenate([m[:, 1:, :], m[:, :1, :]], axis=1))
            o_ref[n, pl.ds(g * (G // 2) + 1, G // 2), 1:1 + W, :] = ms
        else:
            o_ref[n, pl.ds(g * G + 1, G), 1:1 + W, :] = y
        return c

    jax.lax.fori_loop(0, N * ng, body, 0)


def _conv_nhwc(x, w_mat, bn, *, H, W, C, O, K, pool, G):
    N = x.shape[0]
    Ho = (H // 2) if pool else H
    out_shape = (N, Ho + 2, W + 2, O)
    body = functools.partial(_nhwc_body, N=N, H=H, W=W, C=C, O=O, K=K,
                             pool=pool, G=G)
    return pl.pallas_call(
        body,
        in_specs=[
            pl.BlockSpec(x.shape, lambda: (0,) * 4),
            pl.BlockSpec(w_mat.shape, lambda: (0, 0)),
        ] + [pl.BlockSpec((1, O), lambda: (0, 0))] * 4,
        out_specs=pl.BlockSpec(out_shape, lambda: (0,) * 4),
        out_shape=jax.ShapeDtypeStruct(out_shape, jnp.float32),
    )(x, w_mat, *[v[None, :] for v in bn])


def _local_body(*refs, N, stride):
    if stride == 2:
        x00, x01, x10, x11, w_ref, m_ref, p_ref, g_ref, bb_ref, o_ref = refs
        xp = ((x00, x01), (x10, x11))
    else:
        x_ref, w_ref, m_ref, p_ref, g_ref, bb_ref, o_ref = refs
    o_ref[...] = jnp.zeros_like(o_ref)
    w = w_ref[...]
    bn = (m_ref[...], p_ref[...], g_ref[...], bb_ref[...])
    parts = []
    for a in (0, 1, 2):
        for b2 in (0, 1, 2):
            if stride == 2:
                xs = xp[a % 2][b2 % 2][:, a // 2:a // 2 + 7,
                                       b2 // 2:b2 // 2 + 7, :]
            else:
                xs = x_ref[:, a:a + 7, b2:b2 + 7, :]
            parts.append(xs.reshape(N * 49, 1024))
    patch = jnp.concatenate(parts, axis=-1)
    y = jax.lax.dot_general(patch, w, (((1,), (0,)), ((), ())),
                            preferred_element_type=jnp.float32)
    y = _bn_leaky(y, *bn).reshape(N, 7, 7, 1024)
    o_ref[:, 1:8, 1:8, :] = y


def _conv_local(xs, w_mat, bn, *, stride):
    N = xs[0].shape[0]
    out_shape = (N, 9, 9, 1024)
    body = functools.partial(_local_body, N=N, stride=stride)
    in_arrays = list(xs) + [w_mat] + [v[None, :] for v in bn]
    return pl.pallas_call(
        body,
        in_specs=[pl.BlockSpec(a.shape, (lambda nd=a.ndim: (0,) * nd))
                  for a in in_arrays],
        out_specs=pl.BlockSpec(out_shape, lambda: (0,) * 4),
        out_shape=jax.ShapeDtypeStruct(out_shape, jnp.float32),
    )(*in_arrays)


def _fc_body(w_ref, x_ref, b_ref, o_ref, *, nk):
    k = pl.program_id(0)

    @pl.when(k == 0)
    def _():
        o_ref[...] = jnp.zeros_like(o_ref)

    o_ref[...] += jnp.dot(w_ref[...], x_ref[...],
                          preferred_element_type=jnp.float32)

    @pl.when(k == nk - 1)
    def _():
        o_ref[...] = _leaky(o_ref[...] + b_ref[...])


def _fc_reg(w, xT, bias):
    KB = 1024
    nk = w.shape[1] // KB
    return pl.pallas_call(
        functools.partial(_fc_body, nk=nk),
        grid=(nk,),
        in_specs=[
            pl.BlockSpec((4096, KB), lambda k: (0, k)),
            pl.BlockSpec((KB, 8), lambda k: (k, 0)),
            pl.BlockSpec((4096, 1), lambda k: (0, 0)),
        ],
        out_specs=pl.BlockSpec((4096, 8), lambda k: (0, 0)),
        out_shape=jax.ShapeDtypeStruct((4096, 8), jnp.float32),
    )(w, xT, bias)


def _heads_body(h_ref, cw_ref, cb_ref, rw_ref, rb_ref, ow_ref, ob_ref,
                oc_ref, orr_ref, oo_ref):
    h = h_ref[...]
    oc_ref[...] = jnp.dot(cw_ref[...], h,
                          preferred_element_type=jnp.float32) + cb_ref[...]
    orr_ref[...] = jnp.dot(rw_ref[...], h,
                           preferred_element_type=jnp.float32) + rb_ref[...]
    oo_ref[...] = jnp.dot(ow_ref[...], h,
                          preferred_element_type=jnp.float32) + ob_ref[...]


def _heads(h8, cw, cb, rw, rb, ow, ob):
    args = (h8, cw, cb, rw, rb, ow, ob)
    specs = [pl.BlockSpec(a.shape, lambda: (0, 0)) for a in args]
    return pl.pallas_call(
        _heads_body,
        in_specs=specs,
        out_specs=[pl.BlockSpec((cw.shape[0], 8), lambda: (0, 0)),
                   pl.BlockSpec((rw.shape[0], 8), lambda: (0, 0)),
                   pl.BlockSpec((ow.shape[0], 8), lambda: (0, 0))],
        out_shape=[jax.ShapeDtypeStruct((cw.shape[0], 8), jnp.float32),
                   jax.ShapeDtypeStruct((rw.shape[0], 8), jnp.float32),
                   jax.ShapeDtypeStruct((ow.shape[0], 8), jnp.float32)],
    )(*args)


def _bn_params(p):
    return (p['mean'], jnp.sqrt(p['var'] + 1e-5), p['gamma'], p['beta'])


def kernel(x, target, params):
    del target
    N = x.shape[0]
    dk = params['darknet']

    out = jnp.pad(x.transpose(0, 2, 1, 3), ((0, 0), (1, 1), (0, 0), (1, 1)))

    sizes = [448, 224, 112, 112, 112]
    cins = [3, 32, 64, 128, 64]
    for i in range(5):
        O, K, pool = _LAYERS[i]
        C, H = cins[i], sizes[i]
        p = dk[i]
        bn = _bn_params(p)
        if K == 3:
            w_mat = p['w'].transpose(0, 2, 3, 1).reshape(O, 9 * C)
        else:
            w_mat = p['w'].reshape(O, C)
        out = _conv_hcw(out, w_mat, bn, H=H, W=H, C=C, O=O, K=K, pool=pool)
        if pool and i < 4:
            out = jnp.pad(out[:, :, :, 1:1 + H:2],
                          ((0, 0), (0, 0), (0, 0), (1, 1)))
    out = jnp.pad(out[:, :, :, 1:113:2].transpose(0, 1, 3, 2),
                  ((0, 0), (0, 0), (1, 1), (0, 0)))

    sizes = [56, 56, 56, 28, 28, 28, 28, 28, 14, 14, 14, 14, 14]
    cins = [128, 256, 128, 256, 512, 256, 512, 256, 512, 1024, 512, 1024, 512]
    groups = {56: 8, 28: 14, 14: 14}
    for i in range(5, 18):
        O, K, pool = _LAYERS[i]
        C, H = cins[i - 5], sizes[i - 5]
        p = dk[i]
        bn = _bn_params(p)
        if K == 3:
            w_mat = p['w'].transpose(2, 3, 1, 0).reshape(9 * C, O)
        else:
            w_mat = p['w'].reshape(O, C).T
        out = _conv_nhwc(out, w_mat, bn, H=H, W=H, C=C,
                         O=O, K=K, pool=pool, G=groups[H])
        if pool:
            out = jnp.pad(out[:, :, 1:1 + H:2, :],
                          ((0, 0), (0, 0), (1, 1), (0, 0)))

    strides = [2, 1, 1, 1]
    for i in range(4):
        p = params['local'][i]
        bn = _bn_params(p)
        w_mat = p['w'].transpose(2, 3, 1, 0).reshape(9 * 1024, 1024)
        if strides[i] == 2:
            xs = [out[:, pa::2, pb::2, :] for pa in (0, 1) for pb in (0, 1)]
        else:
            xs = [out]
        out = _conv_local(xs, w_mat, bn, stride=strides[i])

    act = out[:, 1:8, 1:8, :]
    flatT = act.transpose(3, 1, 2, 0).reshape(1024 * 49, N)
    flatT8 = jnp.pad(flatT, ((0, 0), (0, 8 - N)))
    h8 = _fc_reg(params['reg_W'], flatT8, params['reg_b'][:, None])
    clsT, respT, offT = _heads(h8,
                               params['cls_W'], params['cls_b'][:, None],
                               params['resp_W'], params['resp_b'][:, None],
                               params['off_W'], params['off_b'][:, None])
    pred_cls = clsT[:, :N].T.reshape(N, _CLS, _S, _S)
    pred_resp = respT[:, :N].T.reshape(N, _BB, _S, _S)
    pred_bbox = offT[:, :N].T.reshape(N, _BB * 4, _S, _S)
    return (pred_cls, pred_resp, pred_bbox)
